# Initial kernel scaffold; baseline (speedup 1.0000x reference)
import jax
import jax.numpy as jnp
from jax.experimental import pallas as pl


def kernel(x, packed_params, w2_block):
    raise NotImplementedError("write your pallas kernel here")


# speedup vs baseline: 1.5514x; 1.5514x over previous
import functools

import jax
import jax.numpy as jnp
from jax.experimental import pallas as pl
from jax.experimental.pallas import tpu as pltpu

_HID = 32
_OUT = 64
_LANES = 128
_EPS = 1e-5
_NC = 2
_TBP = 8192


def _round_up(v, m):
    return (v + m - 1) // m * m


def _xstats_body(xr_ref, o_ref, s_acc, ss_acc, *, nt):
    t = pl.program_id(1)

    @pl.when(t == 0)
    def _():
        s_acc[...] = jnp.zeros_like(s_acc)
        ss_acc[...] = jnp.zeros_like(ss_acc)

    xb = xr_ref[...]
    s_acc[...] += jnp.sum(xb, keepdims=True)
    ss_acc[...] += jnp.sum(xb * xb, keepdims=True)

    @pl.when(t == nt - 1)
    def _():
        lane = jax.lax.broadcasted_iota(jnp.int32, (1, _LANES), 1)
        o_ref[0, 0:1, :] = (jnp.where(lane == 0, s_acc[...], 0.0)
                            + jnp.where(lane == 1, ss_acc[...], 0.0))


def _fold_bn1(pp, pa, inv_n):
    sv = pa[0] + pa[1]
    mean = sv[0:1, 0:1] * inv_n
    var = sv[0:1, 1:2] * inv_n - mean * mean
    w1 = pp[0:1, :]
    g1 = pp[1:2, :]
    be1 = pp[2:3, :]
    s1 = w1 * g1 * jax.lax.rsqrt(var * (w1 * w1) + _EPS)
    t1 = be1 - mean * s1
    return s1, t1


def _hidden(xp, s1, t1):
    he = jnp.maximum(xp[:, 0:1] * s1 + t1, 0.0)
    ho = jnp.maximum(xp[:, 1:2] * s1 + t1, 0.0)
    return jnp.concatenate([he, ho], axis=1)


def _h2stats_body(xp_ref, pp_ref, w2_ref, pa_ref, o_ref, s_acc, ss_acc,
                  *, nt, inv_n, n_rows, need_mask):
    c = pl.program_id(0)
    t = pl.program_id(1)

    @pl.when(t == 0)
    def _():
        s_acc[...] = jnp.zeros_like(s_acc)
        ss_acc[...] = jnp.zeros_like(ss_acc)

    s1, t1 = _fold_bn1(pp_ref[...], pa_ref[...], inv_n)
    xp = xp_ref[...]
    tbp = xp.shape[0]
    hcat = _hidden(xp, s1[0:1, 0:_HID], t1[0:1, 0:_HID])
    if need_mask:
        r = jax.lax.broadcasted_iota(jnp.int32, (tbp, 2 * _HID), 0)
        lane = jax.lax.broadcasted_iota(jnp.int32, (tbp, 2 * _HID), 1)
        orig = 2 * ((c * nt + t) * tbp + r) + (lane >= _HID).astype(jnp.int32)
        hcat = jnp.where(orig < n_rows, hcat, 0.0)
    h2 = jnp.dot(hcat, w2_ref[...], preferred_element_type=jnp.float32)
    s_acc[...] += jnp.sum(h2, axis=0, keepdims=True)
    ss_acc[...] += jnp.sum(h2 * h2, axis=0, keepdims=True)

    @pl.when(t == nt - 1)
    def _():
        o_ref[0, 0:1, :] = s_acc[...]
        o_ref[0, 1:2, :] = ss_acc[...]


def _encode_body(xp_ref, pp_ref, w2_ref, pa_ref, pb_ref, o_ref, coef_ref,
                 *, inv_n):
    t = pl.program_id(1)

    @pl.when(t == 0)
    def _():
        pp = pp_ref[...]
        s1, t1 = _fold_bn1(pp, pa_ref[...], inv_n)
        pb = pb_ref[...]
        sv = pb[0, 0:1, :] + pb[1, 0:1, :]
        ssv = pb[0, 1:2, :] + pb[1, 1:2, :]
        tot_s = sv + pltpu.roll(sv, _OUT, 1)
        tot_ss = ssv + pltpu.roll(ssv, _OUT, 1)
        mu2 = tot_s * inv_n
        var2 = tot_ss * inv_n - mu2 * mu2
        s2 = pp[3:4, :] * jax.lax.rsqrt(var2 + _EPS)
        t2 = pp[4:5, :] - mu2 * s2
        coef_ref[0:1, :] = s1
        coef_ref[1:2, :] = t1
        coef_ref[2:3, :] = s2
        coef_ref[3:4, :] = t2

    hcat = _hidden(xp_ref[...], coef_ref[0:1, 0:_HID], coef_ref[1:2, 0:_HID])
    h2 = jnp.dot(hcat, w2_ref[...], preferred_element_type=jnp.float32)
    o_ref[...] = jnp.maximum(h2 * coef_ref[2:3, :] + coef_ref[3:4, :], 0.0)


def kernel(x, packed_params, w2_block):
    B = x.shape[0]
    bp = _round_up(B, 2 * _TBP * _NC)
    if bp != B:
        x = jnp.pad(x, ((0, bp - B), (0, 0)))
    n2 = bp // 2
    xp = x.reshape(n2, 2)
    nt = n2 // _TBP // _NC
    inv_n = 1.0 / float(B)
    need_mask = bp != B
    sem = pltpu.CompilerParams(dimension_semantics=("parallel", "arbitrary"))

    r_all = bp // _LANES
    nt_a = 4
    r_blk = r_all // _NC // nt_a
    pa = pl.pallas_call(
        functools.partial(_xstats_body, nt=nt_a),
        out_shape=jax.ShapeDtypeStruct((_NC, 1, _LANES), jnp.float32),
        grid=(_NC, nt_a),
        in_specs=[pl.BlockSpec((r_blk, _LANES),
                               lambda c, t, n=nt_a: (c * n + t, 0))],
        out_specs=pl.BlockSpec((1, 1, _LANES), lambda c, t: (c, 0, 0)),
        scratch_shapes=[pltpu.VMEM((1, 1), jnp.float32),
                        pltpu.VMEM((1, 1), jnp.float32)],
        compiler_params=sem,
    )(x.reshape(r_all, _LANES))

    pb = pl.pallas_call(
        functools.partial(_h2stats_body, nt=nt, inv_n=inv_n, n_rows=B,
                          need_mask=need_mask),
        out_shape=jax.ShapeDtypeStruct((_NC, 2, _LANES), jnp.float32),
        grid=(_NC, nt),
        in_specs=[
            pl.BlockSpec((_TBP, 2), lambda c, t, n=nt: (c * n + t, 0)),
            pl.BlockSpec((5, _LANES), lambda c, t: (0, 0)),
            pl.BlockSpec((2 * _HID, _LANES), lambda c, t: (0, 0)),
            pl.BlockSpec((_NC, 1, _LANES), lambda c, t: (0, 0, 0)),
        ],
        out_specs=pl.BlockSpec((1, 2, _LANES), lambda c, t: (c, 0, 0)),
        scratch_shapes=[pltpu.VMEM((1, _LANES), jnp.float32),
                        pltpu.VMEM((1, _LANES), jnp.float32)],
        compiler_params=sem,
    )(xp, packed_params, w2_block, pa)

    out_packed = pl.pallas_call(
        functools.partial(_encode_body, inv_n=inv_n),
        out_shape=jax.ShapeDtypeStruct((n2, _LANES), jnp.float32),
        grid=(_NC, nt),
        in_specs=[
            pl.BlockSpec((_TBP, 2), lambda c, t, n=nt: (c * n + t, 0)),
            pl.BlockSpec((5, _LANES), lambda c, t: (0, 0)),
            pl.BlockSpec((2 * _HID, _LANES), lambda c, t: (0, 0)),
            pl.BlockSpec((_NC, 1, _LANES), lambda c, t: (0, 0, 0)),
            pl.BlockSpec((_NC, 2, _LANES), lambda c, t: (0, 0, 0)),
        ],
        out_specs=pl.BlockSpec((_TBP, _LANES),
                               lambda c, t, n=nt: (c * n + t, 0)),
        scratch_shapes=[pltpu.VMEM((8, _LANES), jnp.float32)],
        compiler_params=sem,
    )(xp, packed_params, w2_block, pa, pb)

    return out_packed.reshape(bp, _OUT)[:B]


# speedup vs baseline: 2.8461x; 1.8345x over previous
import functools

import jax
import jax.numpy as jnp
from jax.experimental import pallas as pl
from jax.experimental.pallas import tpu as pltpu

_HID = 32
_OUT = 64
_LANES = 128
_EPS = 1e-5
_NC = 2
_RT = 64


def _round_up(v, m):
    return (v + m - 1) // m * m


def _xstats_body(xr_ref, o_ref, s_acc, ss_acc, *, nt):
    t = pl.program_id(1)

    @pl.when(t == 0)
    def _():
        s_acc[...] = jnp.zeros_like(s_acc)
        ss_acc[...] = jnp.zeros_like(ss_acc)

    xb = xr_ref[...]
    s_acc[...] += jnp.sum(xb, keepdims=True)
    ss_acc[...] += jnp.sum(xb * xb, keepdims=True)

    @pl.when(t == nt - 1)
    def _():
        lane = jax.lax.broadcasted_iota(jnp.int32, (1, _LANES), 1)
        o_ref[0, 0:1, :] = (jnp.where(lane == 0, s_acc[...], 0.0)
                            + jnp.where(lane == 1, ss_acc[...], 0.0))


def _fold_bn1_cols(ppt, pa, inv_n):
    sv = pa[0] + pa[1]
    mean = sv[0:1, 0:1] * inv_n
    var = sv[0:1, 1:2] * inv_n - mean * mean
    w1c = ppt[0:_HID, 0:1]
    g1c = ppt[0:_HID, 1:2]
    be1c = ppt[0:_HID, 2:3]
    s1c = w1c * g1c * jax.lax.rsqrt(var * (w1c * w1c) + _EPS)
    t1c = be1c - mean * s1c
    return s1c, t1c


def _h2_lane_major(xv, s1c, t1c, w2e):
    h3 = jnp.maximum(xv[:, None, :] * s1c[None, :, :] + t1c[None, :, :], 0.0)
    return jax.lax.dot_general(
        h3, w2e, (((1,), (0,)), ((), ())),
        preferred_element_type=jnp.float32)


def _h2stats_body(xd_ref, ppt_ref, w2_ref, pa_ref, o_ref, s_acc, ss_acc,
                  *, nt, inv_n, n_rows, need_mask):
    c = pl.program_id(0)
    t = pl.program_id(1)

    @pl.when(t == 0)
    def _():
        s_acc[...] = jnp.zeros_like(s_acc)
        ss_acc[...] = jnp.zeros_like(ss_acc)

    s1c, t1c = _fold_bn1_cols(ppt_ref[...], pa_ref[...], inv_n)
    xv = xd_ref[...]
    rt = xv.shape[0]
    h3 = jnp.maximum(xv[:, None, :] * s1c[None, :, :] + t1c[None, :, :], 0.0)
    if need_mask:
        r3 = jax.lax.broadcasted_iota(jnp.int32, (rt, _HID, _LANES), 0)
        l3 = jax.lax.broadcasted_iota(jnp.int32, (rt, _HID, _LANES), 2)
        elem = ((c * nt + t) * rt + r3) * _LANES + l3
        h3 = jnp.where(elem < n_rows, h3, 0.0)
    h2 = jax.lax.dot_general(
        h3, w2_ref[0:_HID, 0:_OUT], (((1,), (0,)), ((), ())),
        preferred_element_type=jnp.float32)
    s_acc[0:1, 0:_OUT] += jnp.sum(h2, axis=(0, 1))[None, :]
    ss_acc[0:1, 0:_OUT] += jnp.sum(h2 * h2, axis=(0, 1))[None, :]

    @pl.when(t == nt - 1)
    def _():
        o_ref[0, 0:1, :] = s_acc[...]
        o_ref[0, 1:2, :] = ss_acc[...]


def _encode_body(xd_ref, pp_ref, ppt_ref, w2_ref, pa_ref, pb_ref, o_ref,
                 coef1_ref, coef2_ref, *, inv_n):
    t = pl.program_id(1)

    @pl.when(t == 0)
    def _():
        s1c, t1c = _fold_bn1_cols(ppt_ref[...], pa_ref[...], inv_n)
        pb = pb_ref[...]
        sv = pb[0, 0:1, :] + pb[1, 0:1, :]
        ssv = pb[0, 1:2, :] + pb[1, 1:2, :]
        mu2 = sv * inv_n
        var2 = ssv * inv_n - mu2 * mu2
        pp = pp_ref[...]
        s2 = pp[3:4, :] * jax.lax.rsqrt(var2 + _EPS)
        t2 = pp[4:5, :] - mu2 * s2
        coef1_ref[0:_HID, 0:1] = s1c
        coef1_ref[0:_HID, 1:2] = t1c
        coef2_ref[0:1, :] = s2
        coef2_ref[1:2, :] = t2

    s1c = coef1_ref[0:_HID, 0:1]
    t1c = coef1_ref[0:_HID, 1:2]
    h2 = _h2_lane_major(xd_ref[...], s1c, t1c, w2_ref[0:_HID, 0:_OUT])
    rt = h2.shape[0]
    s2 = coef2_ref[0:1, 0:_OUT][None, :, :]
    t2 = coef2_ref[1:2, 0:_OUT][None, :, :]
    out3 = jnp.maximum(h2 * s2 + t2, 0.0)
    o_ref[...] = out3.reshape(rt * _LANES, _OUT)


def kernel(x, packed_params, w2_block):
    B = x.shape[0]
    bp = _round_up(B, _RT * _LANES * _NC)
    if bp != B:
        x = jnp.pad(x, ((0, bp - B), (0, 0)))
    inv_n = 1.0 / float(B)
    need_mask = bp != B
    sem = pltpu.CompilerParams(dimension_semantics=("parallel", "arbitrary"))

    r_all = bp // _LANES
    xd = x.reshape(r_all, _LANES)
    ppt = packed_params.T
    nt = r_all // _RT // _NC

    nt_a = 4
    r_blk = r_all // _NC // nt_a
    pa = pl.pallas_call(
        functools.partial(_xstats_body, nt=nt_a),
        out_shape=jax.ShapeDtypeStruct((_NC, 1, _LANES), jnp.float32),
        grid=(_NC, nt_a),
        in_specs=[pl.BlockSpec((r_blk, _LANES),
                               lambda c, t, n=nt_a: (c * n + t, 0))],
        out_specs=pl.BlockSpec((1, 1, _LANES), lambda c, t: (c, 0, 0)),
        scratch_shapes=[pltpu.VMEM((1, 1), jnp.float32),
                        pltpu.VMEM((1, 1), jnp.float32)],
        compiler_params=sem,
    )(xd)

    pb = pl.pallas_call(
        functools.partial(_h2stats_body, nt=nt, inv_n=inv_n, n_rows=B,
                          need_mask=need_mask),
        out_shape=jax.ShapeDtypeStruct((_NC, 2, _LANES), jnp.float32),
        grid=(_NC, nt),
        in_specs=[
            pl.BlockSpec((_RT, _LANES), lambda c, t, n=nt: (c * n + t, 0)),
            pl.BlockSpec((_LANES, 5), lambda c, t: (0, 0)),
            pl.BlockSpec((2 * _HID, _LANES), lambda c, t: (0, 0)),
            pl.BlockSpec((_NC, 1, _LANES), lambda c, t: (0, 0, 0)),
        ],
        out_specs=pl.BlockSpec((1, 2, _LANES), lambda c, t: (c, 0, 0)),
        scratch_shapes=[pltpu.VMEM((1, _LANES), jnp.float32),
                        pltpu.VMEM((1, _LANES), jnp.float32)],
        compiler_params=sem,
    )(xd, ppt, w2_block, pa)

    out = pl.pallas_call(
        functools.partial(_encode_body, inv_n=inv_n),
        out_shape=jax.ShapeDtypeStruct((bp, _OUT), jnp.float32),
        grid=(_NC, nt),
        in_specs=[
            pl.BlockSpec((_RT, _LANES), lambda c, t, n=nt: (c * n + t, 0)),
            pl.BlockSpec((5, _LANES), lambda c, t: (0, 0)),
            pl.BlockSpec((_LANES, 5), lambda c, t: (0, 0)),
            pl.BlockSpec((2 * _HID, _LANES), lambda c, t: (0, 0)),
            pl.BlockSpec((_NC, 1, _LANES), lambda c, t: (0, 0, 0)),
            pl.BlockSpec((_NC, 2, _LANES), lambda c, t: (0, 0, 0)),
        ],
        out_specs=pl.BlockSpec((_RT * _LANES, _OUT),
                               lambda c, t, n=nt: (c * n + t, 0)),
        scratch_shapes=[pltpu.VMEM((_HID, 2), jnp.float32),
                        pltpu.VMEM((2, _LANES), jnp.float32)],
        compiler_params=sem,
    )(xd, packed_params, ppt, w2_block, pa, pb)

    return out[:B]


# speedup vs baseline: 2.8472x; 1.0004x over previous
import functools

import jax
import jax.numpy as jnp
from jax.experimental import pallas as pl
from jax.experimental.pallas import tpu as pltpu

_HID = 32
_OUT = 64
_LANES = 128
_EPS = 1e-5
_NC = 2
_RT = 64


def _round_up(v, m):
    return (v + m - 1) // m * m


def _xstats_body(xr_ref, o_ref, s_acc, ss_acc, *, nt):
    t = pl.program_id(1)

    @pl.when(t == 0)
    def _():
        s_acc[...] = jnp.zeros_like(s_acc)
        ss_acc[...] = jnp.zeros_like(ss_acc)

    xb = xr_ref[...]
    s_acc[...] += jnp.sum(xb, keepdims=True)
    ss_acc[...] += jnp.sum(xb * xb, keepdims=True)

    @pl.when(t == nt - 1)
    def _():
        lane = jax.lax.broadcasted_iota(jnp.int32, (1, _LANES), 1)
        o_ref[0, 0:1, :] = (jnp.where(lane == 0, s_acc[...], 0.0)
                            + jnp.where(lane == 1, ss_acc[...], 0.0))


def _fold_bn1_cols(ppt, pa, inv_n):
    sv = pa[0] + pa[1]
    mean = sv[0:1, 0:1] * inv_n
    var = sv[0:1, 1:2] * inv_n - mean * mean
    w1c = ppt[0:_HID, 0:1]
    g1c = ppt[0:_HID, 1:2]
    be1c = ppt[0:_HID, 2:3]
    s1c = w1c * g1c * jax.lax.rsqrt(var * (w1c * w1c) + _EPS)
    t1c = be1c - mean * s1c
    return s1c, t1c


def _h2_lane_major(xv, s1c, t1c, w2e):
    h3 = jnp.maximum(xv[:, None, :] * s1c[None, :, :] + t1c[None, :, :], 0.0)
    return jax.lax.dot_general(
        h3, w2e, (((1,), (0,)), ((), ())),
        preferred_element_type=jnp.float32)


def _h2stats_body(xd_ref, ppt_ref, w2_ref, pa_ref, o_ref, s_acc, ss_acc,
                  *, nt, inv_n, n_rows, need_mask):
    c = pl.program_id(0)
    t = pl.program_id(1)

    @pl.when(t == 0)
    def _():
        s_acc[...] = jnp.zeros_like(s_acc)
        ss_acc[...] = jnp.zeros_like(ss_acc)

    s1c, t1c = _fold_bn1_cols(ppt_ref[...], pa_ref[...], inv_n)
    xv = xd_ref[...]
    rt = xv.shape[0]
    h3 = jnp.maximum(xv[:, None, :] * s1c[None, :, :] + t1c[None, :, :], 0.0)
    if need_mask:
        r3 = jax.lax.broadcasted_iota(jnp.int32, (rt, _HID, _LANES), 0)
        l3 = jax.lax.broadcasted_iota(jnp.int32, (rt, _HID, _LANES), 2)
        elem = ((c * nt + t) * rt + r3) * _LANES + l3
        h3 = jnp.where(elem < n_rows, h3, 0.0)
    h2 = jax.lax.dot_general(
        h3, w2_ref[0:_HID, 0:_OUT], (((1,), (0,)), ((), ())),
        preferred_element_type=jnp.float32)
    s_acc[0:1, 0:_OUT] += jnp.sum(h2, axis=(0, 1))[None, :]
    ss_acc[0:1, 0:_OUT] += jnp.sum(h2 * h2, axis=(0, 1))[None, :]

    @pl.when(t == nt - 1)
    def _():
        o_ref[0, 0:1, :] = s_acc[...]
        o_ref[0, 1:2, :] = ss_acc[...]


def _encode_body(xd_ref, pp_ref, ppt_ref, w2_ref, pa_ref, pb_ref, o_ref,
                 coef1_ref, coef2_ref, *, inv_n):
    t = pl.program_id(1)

    @pl.when(t == 0)
    def _():
        s1c, t1c = _fold_bn1_cols(ppt_ref[...], pa_ref[...], inv_n)
        pb = pb_ref[...]
        sv = pb[0, 0:1, :] + pb[1, 0:1, :]
        ssv = pb[0, 1:2, :] + pb[1, 1:2, :]
        mu2 = sv * inv_n
        var2 = ssv * inv_n - mu2 * mu2
        pp = pp_ref[...]
        s2 = pp[3:4, :] * jax.lax.rsqrt(var2 + _EPS)
        t2 = pp[4:5, :] - mu2 * s2
        coef1_ref[0:_HID, 0:1] = s1c
        coef1_ref[0:_HID, 1:2] = t1c
        coef2_ref[0:1, :] = s2
        coef2_ref[1:2, :] = t2

    s1c = coef1_ref[0:_HID, 0:1]
    t1c = coef1_ref[0:_HID, 1:2]
    h2 = _h2_lane_major(xd_ref[...], s1c, t1c, w2_ref[0:_HID, 0:_OUT])
    rt = h2.shape[0]
    s2 = coef2_ref[0:1, 0:_OUT][None, :, :]
    t2 = coef2_ref[1:2, 0:_OUT][None, :, :]
    out3 = jnp.maximum(h2 * s2 + t2, 0.0)
    o_ref[...] = out3.reshape(rt * _LANES, _OUT)


def kernel(x, packed_params, w2_block):
    B = x.shape[0]
    bp = _round_up(B, _RT * _LANES * _NC)
    if bp != B:
        x = jnp.pad(x, ((0, bp - B), (0, 0)))
    inv_n = 1.0 / float(B)
    need_mask = bp != B
    sem = pltpu.CompilerParams(dimension_semantics=("arbitrary", "arbitrary"))

    r_all = bp // _LANES
    xd = x.reshape(r_all, _LANES)
    ppt = packed_params.T
    nt = r_all // _RT // _NC

    nt_a = 4
    r_blk = r_all // _NC // nt_a
    pa = pl.pallas_call(
        functools.partial(_xstats_body, nt=nt_a),
        out_shape=jax.ShapeDtypeStruct((_NC, 1, _LANES), jnp.float32),
        grid=(_NC, nt_a),
        in_specs=[pl.BlockSpec((r_blk, _LANES),
                               lambda c, t, n=nt_a: (c * n + t, 0))],
        out_specs=pl.BlockSpec((1, 1, _LANES), lambda c, t: (c, 0, 0)),
        scratch_shapes=[pltpu.VMEM((1, 1), jnp.float32),
                        pltpu.VMEM((1, 1), jnp.float32)],
        compiler_params=sem,
    )(xd)

    pb = pl.pallas_call(
        functools.partial(_h2stats_body, nt=nt, inv_n=inv_n, n_rows=B,
                          need_mask=need_mask),
        out_shape=jax.ShapeDtypeStruct((_NC, 2, _LANES), jnp.float32),
        grid=(_NC, nt),
        in_specs=[
            pl.BlockSpec((_RT, _LANES), lambda c, t, n=nt: (c * n + t, 0)),
            pl.BlockSpec((_LANES, 5), lambda c, t: (0, 0)),
            pl.BlockSpec((2 * _HID, _LANES), lambda c, t: (0, 0)),
            pl.BlockSpec((_NC, 1, _LANES), lambda c, t: (0, 0, 0)),
        ],
        out_specs=pl.BlockSpec((1, 2, _LANES), lambda c, t: (c, 0, 0)),
        scratch_shapes=[pltpu.VMEM((1, _LANES), jnp.float32),
                        pltpu.VMEM((1, _LANES), jnp.float32)],
        compiler_params=sem,
    )(xd, ppt, w2_block, pa)

    out = pl.pallas_call(
        functools.partial(_encode_body, inv_n=inv_n),
        out_shape=jax.ShapeDtypeStruct((bp, _OUT), jnp.float32),
        grid=(_NC, nt),
        in_specs=[
            pl.BlockSpec((_RT, _LANES), lambda c, t, n=nt: (c * n + t, 0)),
            pl.BlockSpec((5, _LANES), lambda c, t: (0, 0)),
            pl.BlockSpec((_LANES, 5), lambda c, t: (0, 0)),
            pl.BlockSpec((2 * _HID, _LANES), lambda c, t: (0, 0)),
            pl.BlockSpec((_NC, 1, _LANES), lambda c, t: (0, 0, 0)),
            pl.BlockSpec((_NC, 2, _LANES), lambda c, t: (0, 0, 0)),
        ],
        out_specs=pl.BlockSpec((_RT * _LANES, _OUT),
                               lambda c, t, n=nt: (c * n + t, 0)),
        scratch_shapes=[pltpu.VMEM((_HID, 2), jnp.float32),
                        pltpu.VMEM((2, _LANES), jnp.float32)],
        compiler_params=sem,
    )(xd, packed_params, ppt, w2_block, pa, pb)

    return out if bp == B else out[:B]
